# baseline (speedup 1.0000x reference)
import functools

import jax
import jax.numpy as jnp
from jax import lax
from jax.experimental import pallas as pl
from jax.experimental.pallas import tpu as pltpu
from jax.experimental.pallas import tpu_sc as plsc

N = 10000
E = 320000
H = 128

_NC = 2
_NS = 16
_NW = _NC * _NS
_NSL = 5
_ESL = E // _NSL
_C = 80
_GCH = _ESL // _NS // _C
_SCH = _ESL // _NW // _C
_NPAD = 10240
_RPT = _NPAD // _NS
_BE = 2000
_NBE = _ESL // _BE

_F32 = jnp.float32


def _ln(x, g, b):
    mu = jnp.mean(x, axis=-1, keepdims=True)
    var = jnp.mean((x - mu) * (x - mu), axis=-1, keepdims=True)
    return (x - mu) * lax.rsqrt(var + 1e-5) * g + b


def _proj_body(n_ref, wa_ref, wb_ref, wc_ref, psr_ref, pc_ref):
    x = n_ref[...]
    psr_ref[0] = jnp.dot(x, wa_ref[...], preferred_element_type=_F32)
    psr_ref[1] = jnp.dot(x, wb_ref[...], preferred_element_type=_F32)
    pc_ref[...] = jnp.dot(x, wc_ref[...], preferred_element_type=_F32)


def _project_nodes(nodes_pad, wa, wb, wc):
    bn = 2048
    grid = (_NPAD // bn,)
    blk = pl.BlockSpec((bn, H), lambda i: (i, 0))
    wblk = pl.BlockSpec((H, H), lambda i: (0, 0))
    return pl.pallas_call(
        _proj_body,
        grid=grid,
        in_specs=[blk, wblk, wblk, wblk],
        out_specs=[pl.BlockSpec((2, bn, H), lambda i: (0, i, 0)), blk],
        out_shape=[jax.ShapeDtypeStruct((2, _NPAD, H), _F32),
                   jax.ShapeDtypeStruct((_NPAD, H), _F32)],
    )(nodes_pad, wa, wb, wc)


def _edge_body(gs_ref, gr_ref, e_ref, w0_ref, w1_ref, w2_ref, w3_ref,
               b0_ref, b1_ref, b2_ref, b3_ref, g_ref, bb_ref, o_ref):
    x = gs_ref[0] + gr_ref[0] + b0_ref[...]
    x = x + jnp.dot(e_ref[...], w0_ref[...], preferred_element_type=_F32)
    x = jnp.maximum(x, 0.0)
    x = jnp.maximum(jnp.dot(x, w1_ref[...], preferred_element_type=_F32) + b1_ref[...], 0.0)
    x = jnp.maximum(jnp.dot(x, w2_ref[...], preferred_element_type=_F32) + b2_ref[...], 0.0)
    x = jnp.dot(x, w3_ref[...], preferred_element_type=_F32) + b3_ref[...]
    o_ref[...] = _ln(x, g_ref[...], bb_ref[...])


def _edge_mlp(s, gsr, edges, w0c, w1, w2, w3, b0, b1, b2, b3, g, bb):
    blk = pl.BlockSpec((_BE, H), lambda i: (i, 0))
    eblk = pl.BlockSpec((_BE, H), lambda i, s=s: (i + s * _NBE, 0))
    gblk0 = pl.BlockSpec((1, _BE, H), lambda i: (0, i, 0))
    gblk1 = pl.BlockSpec((1, _BE, H), lambda i: (1, i, 0))
    wblk = pl.BlockSpec((H, H), lambda i: (0, 0))
    vblk = pl.BlockSpec((1, H), lambda i: (0, 0))
    return pl.pallas_call(
        _edge_body,
        grid=(_NBE,),
        in_specs=[gblk0, gblk1, eblk, wblk, wblk, wblk, wblk,
                  vblk, vblk, vblk, vblk, vblk, vblk],
        out_specs=blk,
        out_shape=jax.ShapeDtypeStruct((_ESL, H), _F32),
    )(gsr, gsr, edges, w0c, w1, w2, w3, b0, b1, b2, b3, g, bb)


def _node_body(q_ref, a0_ref, a1_ref, w0_ref, w1_ref, w2_ref, w3_ref,
               b0_ref, b1_ref, b2_ref, b3_ref, g_ref, bb_ref, o_ref):
    agg = a0_ref[0] + a1_ref[0]
    x = q_ref[...] + jnp.dot(agg, w0_ref[...], preferred_element_type=_F32) + b0_ref[...]
    x = jnp.maximum(x, 0.0)
    x = jnp.maximum(jnp.dot(x, w1_ref[...], preferred_element_type=_F32) + b1_ref[...], 0.0)
    x = jnp.maximum(jnp.dot(x, w2_ref[...], preferred_element_type=_F32) + b2_ref[...], 0.0)
    x = jnp.dot(x, w3_ref[...], preferred_element_type=_F32) + b3_ref[...]
    o_ref[...] = _ln(x, g_ref[...], bb_ref[...])


def _node_mlp(q, partials, w0b, w1, w2, w3, b0, b1, b2, b3, g, bb):
    bn = 2000
    grid = (N // bn,)
    blk = pl.BlockSpec((bn, H), lambda i: (i, 0))
    ablk0 = pl.BlockSpec((1, bn, H), lambda i: (0, i, 0))
    ablk1 = pl.BlockSpec((1, bn, H), lambda i: (1, i, 0))
    wblk = pl.BlockSpec((H, H), lambda i: (0, 0))
    vblk = pl.BlockSpec((1, H), lambda i: (0, 0))
    return pl.pallas_call(
        _node_body,
        grid=grid,
        in_specs=[blk, ablk0, ablk1, wblk, wblk, wblk, wblk,
                  vblk, vblk, vblk, vblk, vblk, vblk],
        out_specs=blk,
        out_shape=jax.ShapeDtypeStruct((N, H), _F32),
    )(q, partials, partials, w0b, w1, w2, w3, b0, b1, b2, b3, g, bb)


def _sc_mesh():
    return plsc.VectorSubcoreMesh(core_axis_name="c", subcore_axis_name="s")


def _pipeline(nch, issue, wait_fill, start_drain, wait_drain):
    npairs = (nch - 1) // 2
    issue(0, 0)

    @pl.loop(0, npairs)
    def _pair(j):
        c = 2 * j

        @pl.when(j > 0)
        def _():
            wait_drain(1)

        issue(c + 1, 1)
        wait_fill(0)
        start_drain(c, 0)
        wait_drain(0)
        issue(c + 2, 0)
        wait_fill(1)
        start_drain(c + 1, 1)

    wait_drain(1)
    wait_fill(0)
    start_drain(2 * npairs, 0)
    if nch % 2 == 0:
        issue(nch - 1, 1)
        wait_drain(0)
        wait_fill(1)
        start_drain(nch - 1, 1)
        wait_drain(1)
    else:
        wait_drain(0)


def _gather_kernel(psr_hbm, srx_hbm, out_hbm, idx_v, tabs, r0, r1, g0, g1, w0, w1):
    cid = lax.axis_index("c")
    sid = lax.axis_index("s")
    rows = (r0, r1)
    gsem = (g0, g1)
    wsem = (w0, w1)
    base = sid * (_GCH * _C)

    pltpu.sync_copy(psr_hbm.at[cid, pl.ds(sid * _RPT, _RPT)],
                    tabs.at[pl.ds(sid * _RPT, _RPT)])
    pltpu.sync_copy(srx_hbm.at[cid, sid], idx_v)
    plsc.subcore_barrier()

    def issue(c, b):
        pltpu.async_copy(tabs.at[idx_v.at[c]], rows[b], gsem[b])

    def wait_fill(b):
        pltpu.make_async_copy(tabs.at[pl.ds(0, _C)], rows[b], gsem[b]).wait()

    def start_drain(c, b):
        pltpu.async_copy(rows[b], out_hbm.at[cid, pl.ds(base + c * _C, _C)],
                         wsem[b])

    def wait_drain(b):
        pltpu.make_async_copy(rows[b], out_hbm.at[cid, pl.ds(0, _C)],
                              wsem[b]).wait()

    _pipeline(_GCH, issue, wait_fill, start_drain, wait_drain)


def _sc_gather(psr, srx_s):
    k = pl.kernel(
        _gather_kernel,
        out_type=jax.ShapeDtypeStruct((2, _ESL, H), _F32),
        mesh=_sc_mesh(),
        scratch_types=(
            [pltpu.VMEM((_GCH, _C), jnp.int32)]
            + [pltpu.VMEM_SHARED((_NPAD, H), _F32)]
            + [pltpu.VMEM((_C, H), _F32)] * 2
            + [pltpu.SemaphoreType.DMA] * 4
        ),
    )
    return k(psr, srx_s)


def _scatter_kernel(ne_hbm, r_hbm, init_hbm, out_hbm, *scr):
    idx_v = scr[0]
    rows = (scr[1], scr[2])
    acc = scr[3]
    lsem = (scr[4], scr[5])
    ssem = (scr[6], scr[7])

    cid = lax.axis_index("c")
    sid = lax.axis_index("s")
    wid = sid * _NC + cid
    base = wid * (_SCH * _C)

    pltpu.sync_copy(init_hbm.at[cid, pl.ds(sid * _RPT, _RPT)],
                    acc.at[pl.ds(sid * _RPT, _RPT)])
    pltpu.sync_copy(r_hbm.at[wid], idx_v)
    plsc.subcore_barrier()

    def load(c, b):
        pltpu.async_copy(ne_hbm.at[pl.ds(base + c * _C, _C)], rows[b], lsem[b])

    def wait_load(b):
        pltpu.make_async_copy(ne_hbm.at[pl.ds(0, _C)], rows[b], lsem[b]).wait()

    def start_sc(c, b):
        pltpu.async_copy(rows[b], acc.at[idx_v.at[c]], ssem[b], add=True)

    def wait_sc(b):
        pltpu.make_async_copy(rows[b], acc.at[idx_v.at[0]], ssem[b]).wait()

    _pipeline(_SCH, load, wait_load, start_sc, wait_sc)

    plsc.subcore_barrier()
    pltpu.sync_copy(acc.at[pl.ds(sid * _RPT, _RPT)],
                    out_hbm.at[cid, pl.ds(sid * _RPT, _RPT)])


def _sc_scatter(ne_s, r2d_s, init):
    k = pl.kernel(
        _scatter_kernel,
        out_type=jax.ShapeDtypeStruct((_NC, _NPAD, H), _F32),
        mesh=_sc_mesh(),
        scratch_types=(
            [pltpu.VMEM((_SCH, _C), jnp.int32)]
            + [pltpu.VMEM((_C, H), _F32)] * 2
            + [pltpu.VMEM_SHARED((_NPAD, H), _F32)]
            + [pltpu.SemaphoreType.DMA] * 4
        ),
    )
    return k(ne_s, r2d_s, init)


def kernel(nodes, edges, senders, receivers,
           ew0, eb0, ew1, eb1, ew2, eb2, ew3, eb3, eg, ebeta,
           nw0, nb0, nw1, nb1, nw2, nb2, nw3, nb3, ng, nbeta):
    s32 = senders.astype(jnp.int32)
    r32 = receivers.astype(jnp.int32)
    srx = jnp.stack([s32, r32]).reshape(2, _NSL, _NS, _GCH, _C)
    r2d = r32.reshape(_NSL, _NW, _SCH, _C)

    nodes_pad = jnp.pad(nodes, ((0, _NPAD - N), (0, 0)))
    psr, qn = _project_nodes(nodes_pad, ew0[:H], ew0[H:2 * H], nw0[:H])

    row = lambda v: v.reshape(1, H)
    ebias = (row(eb0), row(eb1), row(eb2), row(eb3), row(eg), row(ebeta))

    nes = []
    partials = jnp.zeros((_NC, _NPAD, H), _F32)
    for s in range(_NSL):
        gsr = _sc_gather(psr, srx[:, s])
        nes.append(_edge_mlp(s, gsr, edges, ew0[2 * H:], ew1, ew2, ew3, *ebias))
        partials = _sc_scatter(nes[s], r2d[s], partials)

    new_edges = jnp.concatenate(nes, axis=0)
    new_nodes = _node_mlp(qn, partials, nw0[H:], nw1, nw2, nw3,
                          row(nb0), row(nb1), row(nb2), row(nb3),
                          row(ng), row(nbeta))
    return (new_nodes, new_edges)


# speedup vs baseline: 1.0592x; 1.0592x over previous
import functools

import jax
import jax.numpy as jnp
from jax import lax
from jax.experimental import pallas as pl
from jax.experimental.pallas import tpu as pltpu
from jax.experimental.pallas import tpu_sc as plsc

N = 10000
E = 320000
H = 128

_NC = 2
_NS = 16
_NW = _NC * _NS
_NSL = 5
_ESL = E // _NSL
_C = 80
_GCH = _ESL // _NS // _C
_SCH = _ESL // _NW // _C
_NPAD = 10240
_RPT = _NPAD // _NS
_BE = 2000
_NBE = _ESL // _BE

_F32 = jnp.float32


def _ln(x, g, b):
    mu = jnp.mean(x, axis=-1, keepdims=True)
    var = jnp.mean((x - mu) * (x - mu), axis=-1, keepdims=True)
    return (x - mu) * lax.rsqrt(var + 1e-5) * g + b


def _proj_body(n_ref, wa_ref, wb_ref, wc_ref, psr_ref, pc_ref):
    x = n_ref[...]
    psr_ref[0] = jnp.dot(x, wa_ref[...], preferred_element_type=_F32)
    psr_ref[1] = jnp.dot(x, wb_ref[...], preferred_element_type=_F32)
    pc_ref[...] = jnp.dot(x, wc_ref[...], preferred_element_type=_F32)


def _project_nodes(nodes_pad, wa, wb, wc):
    bn = 2048
    grid = (_NPAD // bn,)
    blk = pl.BlockSpec((bn, H), lambda i: (i, 0))
    wblk = pl.BlockSpec((H, H), lambda i: (0, 0))
    return pl.pallas_call(
        _proj_body,
        grid=grid,
        in_specs=[blk, wblk, wblk, wblk],
        out_specs=[pl.BlockSpec((2, bn, H), lambda i: (0, i, 0)), blk],
        out_shape=[jax.ShapeDtypeStruct((2, _NPAD, H), _F32),
                   jax.ShapeDtypeStruct((_NPAD, H), _F32)],
    )(nodes_pad, wa, wb, wc)


def _edge_body(gs_ref, gr_ref, e_ref, w0_ref, w1_ref, w2_ref, w3_ref,
               b0_ref, b1_ref, b2_ref, b3_ref, g_ref, bb_ref, *rest):
    ofull_ref, oslice_ref = rest[-2], rest[-1]
    x = gs_ref[0] + gr_ref[0] + b0_ref[...]
    x = x + jnp.dot(e_ref[...], w0_ref[...], preferred_element_type=_F32)
    x = jnp.maximum(x, 0.0)
    x = jnp.maximum(jnp.dot(x, w1_ref[...], preferred_element_type=_F32) + b1_ref[...], 0.0)
    x = jnp.maximum(jnp.dot(x, w2_ref[...], preferred_element_type=_F32) + b2_ref[...], 0.0)
    x = jnp.dot(x, w3_ref[...], preferred_element_type=_F32) + b3_ref[...]
    y = _ln(x, g_ref[...], bb_ref[...])
    ofull_ref[...] = y
    oslice_ref[...] = y


def _edge_mlp(s, gsr, edges, ne_prev, w0c, w1, w2, w3, b0, b1, b2, b3, g, bb):
    blk = pl.BlockSpec((_BE, H), lambda i: (i, 0))
    fblk = pl.BlockSpec((_BE, H), lambda i, s=s: (i + s * _NBE, 0))
    gblk0 = pl.BlockSpec((1, _BE, H), lambda i: (0, i, 0))
    gblk1 = pl.BlockSpec((1, _BE, H), lambda i: (1, i, 0))
    wblk = pl.BlockSpec((H, H), lambda i: (0, 0))
    vblk = pl.BlockSpec((1, H), lambda i: (0, 0))
    ins = [gsr, gsr, edges, w0c, w1, w2, w3, b0, b1, b2, b3, g, bb]
    specs = [gblk0, gblk1, fblk, wblk, wblk, wblk, wblk,
             vblk, vblk, vblk, vblk, vblk, vblk]
    aliases = {}
    if ne_prev is not None:
        ins.append(ne_prev)
        specs.append(pl.BlockSpec(memory_space=pl.ANY))
        aliases = {13: 0}
    return pl.pallas_call(
        _edge_body,
        grid=(_NBE,),
        in_specs=specs,
        out_specs=[fblk, blk],
        out_shape=[jax.ShapeDtypeStruct((E, H), _F32),
                   jax.ShapeDtypeStruct((_ESL, H), _F32)],
        input_output_aliases=aliases,
    )(*ins)


def _node_body(q_ref, a0_ref, a1_ref, w0_ref, w1_ref, w2_ref, w3_ref,
               b0_ref, b1_ref, b2_ref, b3_ref, g_ref, bb_ref, o_ref):
    agg = a0_ref[0] + a1_ref[0]
    x = q_ref[...] + jnp.dot(agg, w0_ref[...], preferred_element_type=_F32) + b0_ref[...]
    x = jnp.maximum(x, 0.0)
    x = jnp.maximum(jnp.dot(x, w1_ref[...], preferred_element_type=_F32) + b1_ref[...], 0.0)
    x = jnp.maximum(jnp.dot(x, w2_ref[...], preferred_element_type=_F32) + b2_ref[...], 0.0)
    x = jnp.dot(x, w3_ref[...], preferred_element_type=_F32) + b3_ref[...]
    o_ref[...] = _ln(x, g_ref[...], bb_ref[...])


def _node_mlp(q, partials, w0b, w1, w2, w3, b0, b1, b2, b3, g, bb):
    bn = 2000
    grid = (N // bn,)
    blk = pl.BlockSpec((bn, H), lambda i: (i, 0))
    ablk0 = pl.BlockSpec((1, bn, H), lambda i: (0, i, 0))
    ablk1 = pl.BlockSpec((1, bn, H), lambda i: (1, i, 0))
    wblk = pl.BlockSpec((H, H), lambda i: (0, 0))
    vblk = pl.BlockSpec((1, H), lambda i: (0, 0))
    return pl.pallas_call(
        _node_body,
        grid=grid,
        in_specs=[blk, ablk0, ablk1, wblk, wblk, wblk, wblk,
                  vblk, vblk, vblk, vblk, vblk, vblk],
        out_specs=blk,
        out_shape=jax.ShapeDtypeStruct((N, H), _F32),
    )(q, partials, partials, w0b, w1, w2, w3, b0, b1, b2, b3, g, bb)


def _sc_mesh():
    return plsc.VectorSubcoreMesh(core_axis_name="c", subcore_axis_name="s")


def _pipeline(nch, issue, wait_fill, start_drain, wait_drain):
    npairs = (nch - 1) // 2
    issue(0, 0)

    @pl.loop(0, npairs)
    def _pair(j):
        c = 2 * j

        @pl.when(j > 0)
        def _():
            wait_drain(1)

        issue(c + 1, 1)
        wait_fill(0)
        start_drain(c, 0)
        wait_drain(0)
        issue(c + 2, 0)
        wait_fill(1)
        start_drain(c + 1, 1)

    wait_drain(1)
    wait_fill(0)
    start_drain(2 * npairs, 0)
    if nch % 2 == 0:
        issue(nch - 1, 1)
        wait_drain(0)
        wait_fill(1)
        start_drain(nch - 1, 1)
        wait_drain(1)
    else:
        wait_drain(0)


def _gather_kernel(psr_hbm, srx_hbm, out_hbm, idx_v, tabs, r0, r1, g0, g1, w0, w1):
    cid = lax.axis_index("c")
    sid = lax.axis_index("s")
    rows = (r0, r1)
    gsem = (g0, g1)
    wsem = (w0, w1)
    base = sid * (_GCH * _C)

    pltpu.sync_copy(psr_hbm.at[cid, pl.ds(sid * _RPT, _RPT)],
                    tabs.at[pl.ds(sid * _RPT, _RPT)])
    pltpu.sync_copy(srx_hbm.at[cid, sid], idx_v)
    plsc.subcore_barrier()

    def issue(c, b):
        pltpu.async_copy(tabs.at[idx_v.at[c]], rows[b], gsem[b])

    def wait_fill(b):
        pltpu.make_async_copy(tabs.at[pl.ds(0, _C)], rows[b], gsem[b]).wait()

    def start_drain(c, b):
        pltpu.async_copy(rows[b], out_hbm.at[cid, pl.ds(base + c * _C, _C)],
                         wsem[b])

    def wait_drain(b):
        pltpu.make_async_copy(rows[b], out_hbm.at[cid, pl.ds(0, _C)],
                              wsem[b]).wait()

    _pipeline(_GCH, issue, wait_fill, start_drain, wait_drain)


def _sc_gather(psr, srx_s):
    k = pl.kernel(
        _gather_kernel,
        out_type=jax.ShapeDtypeStruct((2, _ESL, H), _F32),
        mesh=_sc_mesh(),
        scratch_types=(
            [pltpu.VMEM((_GCH, _C), jnp.int32)]
            + [pltpu.VMEM_SHARED((_NPAD, H), _F32)]
            + [pltpu.VMEM((_C, H), _F32)] * 2
            + [pltpu.SemaphoreType.DMA] * 4
        ),
    )
    return k(psr, srx_s)


def _scatter_kernel(ne_hbm, r_hbm, init_hbm, out_hbm, *scr):
    idx_v = scr[0]
    rows = (scr[1], scr[2])
    acc = scr[3]
    lsem = (scr[4], scr[5])
    ssem = (scr[6], scr[7])

    cid = lax.axis_index("c")
    sid = lax.axis_index("s")
    wid = sid * _NC + cid
    base = wid * (_SCH * _C)

    pltpu.sync_copy(init_hbm.at[cid, pl.ds(sid * _RPT, _RPT)],
                    acc.at[pl.ds(sid * _RPT, _RPT)])
    pltpu.sync_copy(r_hbm.at[wid], idx_v)
    plsc.subcore_barrier()

    def load(c, b):
        pltpu.async_copy(ne_hbm.at[pl.ds(base + c * _C, _C)], rows[b], lsem[b])

    def wait_load(b):
        pltpu.make_async_copy(ne_hbm.at[pl.ds(0, _C)], rows[b], lsem[b]).wait()

    def start_sc(c, b):
        pltpu.async_copy(rows[b], acc.at[idx_v.at[c]], ssem[b], add=True)

    def wait_sc(b):
        pltpu.make_async_copy(rows[b], acc.at[idx_v.at[0]], ssem[b]).wait()

    _pipeline(_SCH, load, wait_load, start_sc, wait_sc)

    plsc.subcore_barrier()
    pltpu.sync_copy(acc.at[pl.ds(sid * _RPT, _RPT)],
                    out_hbm.at[cid, pl.ds(sid * _RPT, _RPT)])


def _sc_scatter(ne_s, r2d_s, init):
    k = pl.kernel(
        _scatter_kernel,
        out_type=jax.ShapeDtypeStruct((_NC, _NPAD, H), _F32),
        mesh=_sc_mesh(),
        scratch_types=(
            [pltpu.VMEM((_SCH, _C), jnp.int32)]
            + [pltpu.VMEM((_C, H), _F32)] * 2
            + [pltpu.VMEM_SHARED((_NPAD, H), _F32)]
            + [pltpu.SemaphoreType.DMA] * 4
        ),
    )
    return k(ne_s, r2d_s, init)


def kernel(nodes, edges, senders, receivers,
           ew0, eb0, ew1, eb1, ew2, eb2, ew3, eb3, eg, ebeta,
           nw0, nb0, nw1, nb1, nw2, nb2, nw3, nb3, ng, nbeta):
    s32 = senders.astype(jnp.int32)
    r32 = receivers.astype(jnp.int32)
    srx = jnp.stack([s32, r32]).reshape(2, _NSL, _NS, _GCH, _C)
    r2d = r32.reshape(_NSL, _NW, _SCH, _C)

    nodes_pad = jnp.pad(nodes, ((0, _NPAD - N), (0, 0)))
    psr, qn = _project_nodes(nodes_pad, ew0[:H], ew0[H:2 * H], nw0[:H])

    row = lambda v: v.reshape(1, H)
    ebias = (row(eb0), row(eb1), row(eb2), row(eb3), row(eg), row(ebeta))

    partials = jnp.zeros((_NC, _NPAD, H), _F32)
    ne_full = None
    for s in range(_NSL):
        gsr = _sc_gather(psr, srx[:, s])
        ne_full, ne_s = _edge_mlp(s, gsr, edges, ne_full,
                                  ew0[2 * H:], ew1, ew2, ew3, *ebias)
        partials = _sc_scatter(ne_s, r2d[s], partials)

    new_edges = ne_full
    new_nodes = _node_mlp(qn, partials, nw0[H:], nw1, nw2, nw3,
                          row(nb0), row(nb1), row(nb2), row(nb3),
                          row(ng), row(nbeta))
    return (new_nodes, new_edges)


# speedup vs baseline: 1.0621x; 1.0028x over previous
import functools

import jax
import jax.numpy as jnp
from jax import lax
from jax.experimental import pallas as pl
from jax.experimental.pallas import tpu as pltpu
from jax.experimental.pallas import tpu_sc as plsc

N = 10000
E = 320000
H = 128

_NC = 2
_NS = 16
_NW = _NC * _NS
_NSL = 5
_ESL = E // _NSL
_C = 80
_GCH = _ESL // _NS // _C
_SCH = _ESL // _NW // _C
_NPAD = 10240
_RPT = _NPAD // _NS
_BE = 4000
_NBE = _ESL // _BE

_F32 = jnp.float32


def _ln(x, g, b):
    mu = jnp.mean(x, axis=-1, keepdims=True)
    var = jnp.mean((x - mu) * (x - mu), axis=-1, keepdims=True)
    return (x - mu) * lax.rsqrt(var + 1e-5) * g + b


def _proj_body(n_ref, wa_ref, wb_ref, wc_ref, psr_ref, pc_ref):
    x = n_ref[...]
    psr_ref[0] = jnp.dot(x, wa_ref[...], preferred_element_type=_F32)
    psr_ref[1] = jnp.dot(x, wb_ref[...], preferred_element_type=_F32)
    pc_ref[...] = jnp.dot(x, wc_ref[...], preferred_element_type=_F32)


def _project_nodes(nodes_pad, wa, wb, wc):
    bn = 2048
    grid = (_NPAD // bn,)
    blk = pl.BlockSpec((bn, H), lambda i: (i, 0))
    wblk = pl.BlockSpec((H, H), lambda i: (0, 0))
    return pl.pallas_call(
        _proj_body,
        grid=grid,
        in_specs=[blk, wblk, wblk, wblk],
        out_specs=[pl.BlockSpec((2, bn, H), lambda i: (0, i, 0)), blk],
        out_shape=[jax.ShapeDtypeStruct((2, _NPAD, H), _F32),
                   jax.ShapeDtypeStruct((_NPAD, H), _F32)],
    )(nodes_pad, wa, wb, wc)


def _edge_body(gs_ref, gr_ref, e_ref, w0_ref, w1_ref, w2_ref, w3_ref,
               b0_ref, b1_ref, b2_ref, b3_ref, g_ref, bb_ref, *rest):
    ofull_ref, oslice_ref = rest[-2], rest[-1]
    x = gs_ref[0] + gr_ref[0] + b0_ref[...]
    x = x + jnp.dot(e_ref[...], w0_ref[...], preferred_element_type=_F32)
    x = jnp.maximum(x, 0.0)
    x = jnp.maximum(jnp.dot(x, w1_ref[...], preferred_element_type=_F32) + b1_ref[...], 0.0)
    x = jnp.maximum(jnp.dot(x, w2_ref[...], preferred_element_type=_F32) + b2_ref[...], 0.0)
    x = jnp.dot(x, w3_ref[...], preferred_element_type=_F32) + b3_ref[...]
    y = _ln(x, g_ref[...], bb_ref[...])
    ofull_ref[...] = y
    oslice_ref[...] = y


def _edge_mlp(s, gsr, edges, ne_prev, w0c, w1, w2, w3, b0, b1, b2, b3, g, bb):
    blk = pl.BlockSpec((_BE, H), lambda i: (i, 0))
    fblk = pl.BlockSpec((_BE, H), lambda i, s=s: (i + s * _NBE, 0))
    gblk0 = pl.BlockSpec((1, _BE, H), lambda i: (0, i, 0))
    gblk1 = pl.BlockSpec((1, _BE, H), lambda i: (1, i, 0))
    wblk = pl.BlockSpec((H, H), lambda i: (0, 0))
    vblk = pl.BlockSpec((1, H), lambda i: (0, 0))
    ins = [gsr, gsr, edges, w0c, w1, w2, w3, b0, b1, b2, b3, g, bb]
    specs = [gblk0, gblk1, fblk, wblk, wblk, wblk, wblk,
             vblk, vblk, vblk, vblk, vblk, vblk]
    aliases = {}
    if ne_prev is not None:
        ins.append(ne_prev)
        specs.append(pl.BlockSpec(memory_space=pl.ANY))
        aliases = {13: 0}
    return pl.pallas_call(
        _edge_body,
        grid=(_NBE,),
        in_specs=specs,
        out_specs=[fblk, blk],
        out_shape=[jax.ShapeDtypeStruct((E, H), _F32),
                   jax.ShapeDtypeStruct((_ESL, H), _F32)],
        input_output_aliases=aliases,
    )(*ins)


def _node_body(q_ref, a0_ref, a1_ref, w0_ref, w1_ref, w2_ref, w3_ref,
               b0_ref, b1_ref, b2_ref, b3_ref, g_ref, bb_ref, o_ref):
    agg = a0_ref[0] + a1_ref[0]
    x = q_ref[...] + jnp.dot(agg, w0_ref[...], preferred_element_type=_F32) + b0_ref[...]
    x = jnp.maximum(x, 0.0)
    x = jnp.maximum(jnp.dot(x, w1_ref[...], preferred_element_type=_F32) + b1_ref[...], 0.0)
    x = jnp.maximum(jnp.dot(x, w2_ref[...], preferred_element_type=_F32) + b2_ref[...], 0.0)
    x = jnp.dot(x, w3_ref[...], preferred_element_type=_F32) + b3_ref[...]
    o_ref[...] = _ln(x, g_ref[...], bb_ref[...])


def _node_mlp(q, partials, w0b, w1, w2, w3, b0, b1, b2, b3, g, bb):
    bn = 2000
    grid = (N // bn,)
    blk = pl.BlockSpec((bn, H), lambda i: (i, 0))
    ablk0 = pl.BlockSpec((1, bn, H), lambda i: (0, i, 0))
    ablk1 = pl.BlockSpec((1, bn, H), lambda i: (1, i, 0))
    wblk = pl.BlockSpec((H, H), lambda i: (0, 0))
    vblk = pl.BlockSpec((1, H), lambda i: (0, 0))
    return pl.pallas_call(
        _node_body,
        grid=grid,
        in_specs=[blk, ablk0, ablk1, wblk, wblk, wblk, wblk,
                  vblk, vblk, vblk, vblk, vblk, vblk],
        out_specs=blk,
        out_shape=jax.ShapeDtypeStruct((N, H), _F32),
    )(q, partials, partials, w0b, w1, w2, w3, b0, b1, b2, b3, g, bb)


def _sc_mesh():
    return plsc.VectorSubcoreMesh(core_axis_name="c", subcore_axis_name="s")


def _pipeline(nch, issue, wait_fill, start_drain, wait_drain):
    npairs = (nch - 1) // 2
    issue(0, 0)

    @pl.loop(0, npairs)
    def _pair(j):
        c = 2 * j

        @pl.when(j > 0)
        def _():
            wait_drain(1)

        issue(c + 1, 1)
        wait_fill(0)
        start_drain(c, 0)
        wait_drain(0)
        issue(c + 2, 0)
        wait_fill(1)
        start_drain(c + 1, 1)

    wait_drain(1)
    wait_fill(0)
    start_drain(2 * npairs, 0)
    if nch % 2 == 0:
        issue(nch - 1, 1)
        wait_drain(0)
        wait_fill(1)
        start_drain(nch - 1, 1)
        wait_drain(1)
    else:
        wait_drain(0)


def _gather_kernel(psr_hbm, srx_hbm, out_hbm, idx_v, tabs, r0, r1, g0, g1, w0, w1):
    cid = lax.axis_index("c")
    sid = lax.axis_index("s")
    rows = (r0, r1)
    gsem = (g0, g1)
    wsem = (w0, w1)
    base = sid * (_GCH * _C)

    pltpu.sync_copy(psr_hbm.at[cid, pl.ds(sid * _RPT, _RPT)],
                    tabs.at[pl.ds(sid * _RPT, _RPT)])
    pltpu.sync_copy(srx_hbm.at[cid, sid], idx_v)
    plsc.subcore_barrier()

    def issue(c, b):
        pltpu.async_copy(tabs.at[idx_v.at[c]], rows[b], gsem[b])

    def wait_fill(b):
        pltpu.make_async_copy(tabs.at[pl.ds(0, _C)], rows[b], gsem[b]).wait()

    def start_drain(c, b):
        pltpu.async_copy(rows[b], out_hbm.at[cid, pl.ds(base + c * _C, _C)],
                         wsem[b])

    def wait_drain(b):
        pltpu.make_async_copy(rows[b], out_hbm.at[cid, pl.ds(0, _C)],
                              wsem[b]).wait()

    _pipeline(_GCH, issue, wait_fill, start_drain, wait_drain)


def _sc_gather(psr, srx_s):
    k = pl.kernel(
        _gather_kernel,
        out_type=jax.ShapeDtypeStruct((2, _ESL, H), _F32),
        mesh=_sc_mesh(),
        scratch_types=(
            [pltpu.VMEM((_GCH, _C), jnp.int32)]
            + [pltpu.VMEM_SHARED((_NPAD, H), _F32)]
            + [pltpu.VMEM((_C, H), _F32)] * 2
            + [pltpu.SemaphoreType.DMA] * 4
        ),
    )
    return k(psr, srx_s)


def _scatter_kernel(ne_hbm, r_hbm, init_hbm, out_hbm, *scr):
    idx_v = scr[0]
    rows = (scr[1], scr[2])
    acc = scr[3]
    lsem = (scr[4], scr[5])
    ssem = (scr[6], scr[7])

    cid = lax.axis_index("c")
    sid = lax.axis_index("s")
    wid = sid * _NC + cid
    base = wid * (_SCH * _C)

    pltpu.sync_copy(init_hbm.at[cid, pl.ds(sid * _RPT, _RPT)],
                    acc.at[pl.ds(sid * _RPT, _RPT)])
    pltpu.sync_copy(r_hbm.at[wid], idx_v)
    plsc.subcore_barrier()

    def load(c, b):
        pltpu.async_copy(ne_hbm.at[pl.ds(base + c * _C, _C)], rows[b], lsem[b])

    def wait_load(b):
        pltpu.make_async_copy(ne_hbm.at[pl.ds(0, _C)], rows[b], lsem[b]).wait()

    def start_sc(c, b):
        pltpu.async_copy(rows[b], acc.at[idx_v.at[c]], ssem[b], add=True)

    def wait_sc(b):
        pltpu.make_async_copy(rows[b], acc.at[idx_v.at[0]], ssem[b]).wait()

    _pipeline(_SCH, load, wait_load, start_sc, wait_sc)

    plsc.subcore_barrier()
    pltpu.sync_copy(acc.at[pl.ds(sid * _RPT, _RPT)],
                    out_hbm.at[cid, pl.ds(sid * _RPT, _RPT)])


def _sc_scatter(ne_s, r2d_s, init):
    k = pl.kernel(
        _scatter_kernel,
        out_type=jax.ShapeDtypeStruct((_NC, _NPAD, H), _F32),
        mesh=_sc_mesh(),
        scratch_types=(
            [pltpu.VMEM((_SCH, _C), jnp.int32)]
            + [pltpu.VMEM((_C, H), _F32)] * 2
            + [pltpu.VMEM_SHARED((_NPAD, H), _F32)]
            + [pltpu.SemaphoreType.DMA] * 4
        ),
    )
    return k(ne_s, r2d_s, init)


def kernel(nodes, edges, senders, receivers,
           ew0, eb0, ew1, eb1, ew2, eb2, ew3, eb3, eg, ebeta,
           nw0, nb0, nw1, nb1, nw2, nb2, nw3, nb3, ng, nbeta):
    s32 = senders.astype(jnp.int32)
    r32 = receivers.astype(jnp.int32)
    srx = jnp.stack([s32, r32]).reshape(2, _NSL, _NS, _GCH, _C)
    r2d = r32.reshape(_NSL, _NW, _SCH, _C)

    nodes_pad = jnp.pad(nodes, ((0, _NPAD - N), (0, 0)))
    psr, qn = _project_nodes(nodes_pad, ew0[:H], ew0[H:2 * H], nw0[:H])

    row = lambda v: v.reshape(1, H)
    ebias = (row(eb0), row(eb1), row(eb2), row(eb3), row(eg), row(ebeta))

    partials = jnp.zeros((_NC, _NPAD, H), _F32)
    ne_full = None
    for s in range(_NSL):
        gsr = _sc_gather(psr, srx[:, s])
        ne_full, ne_s = _edge_mlp(s, gsr, edges, ne_full,
                                  ew0[2 * H:], ew1, ew2, ew3, *ebias)
        partials = _sc_scatter(ne_s, r2d[s], partials)

    new_edges = ne_full
    new_nodes = _node_mlp(qn, partials, nw0[H:], nw1, nw2, nw3,
                          row(nb0), row(nb1), row(nb2), row(nb3),
                          row(ng), row(nbeta))
    return (new_nodes, new_edges)
